# baseline (speedup 1.0000x reference)
import functools

import jax
import jax.numpy as jnp
from jax import lax
from jax.experimental import pallas as pl
from jax.experimental.pallas import tpu as pltpu
from jax.experimental.pallas import tpu_sc as plsc

LANES = 16
NC = 2
NS = 16
NW = NC * NS
NBUF = 4
DG = 2


def _iota16():
    return lax.iota(jnp.int32, LANES)


@functools.lru_cache(maxsize=None)
def _make_detile(vocab, embed):
    TCOLS = vocab // 128
    TAIL = vocab % 128
    assert embed == 32 and TAIL % 4 == 0
    mesh = plsc.VectorSubcoreMesh(core_axis_name="c", subcore_axis_name="s")

    @functools.partial(
        pl.kernel,
        mesh=mesh,
        compiler_params=pltpu.CompilerParams(use_tc_tiling_on_sc=False, needs_layout_passes=False),
        out_type=jax.ShapeDtypeStruct((vocab * embed // 128, 128), jnp.float32),
        scratch_types=(
            [pltpu.VMEM((32, 128), jnp.float32) for _ in range(2 * NBUF)]
            + ([pltpu.VMEM((32, TAIL), jnp.float32),
                pltpu.VMEM((TAIL * 32 // 128, 128), jnp.float32)] if TAIL else [])
            + [pltpu.SemaphoreType.DMA for _ in range(2 * NBUF)]
        ),
    )
    def detile(tokT_hbm, out_hbm, *rest):
        vin = rest[:NBUF]
        vout = rest[NBUF:2 * NBUF]
        ntail = 2 if TAIL else 0
        if TAIL:
            tin, tout = rest[2 * NBUF:2 * NBUF + 2]
        gsems = rest[2 * NBUF + ntail:3 * NBUF + ntail]
        ssems = rest[3 * NBUF + ntail:4 * NBUF + ntail]

        wid = lax.axis_index("s") * NC + lax.axis_index("c")
        iota = _iota16()

        def col_of(k):
            return k * NW + wid

        def start_read(k, b):
            pltpu.make_async_copy(
                tokT_hbm.at[:, pl.ds(col_of(k) * 128, 128)], vin[b], gsems[b]
            ).start()

        def wait_read(b):
            pltpu.make_async_copy(
                tokT_hbm.at[:, pl.ds(0, 128)], vin[b], gsems[b]
            ).wait()

        def start_write(k, b):
            pltpu.make_async_copy(
                vout[b], out_hbm.at[pl.ds(col_of(k) * 32, 32)], ssems[b]
            ).start()

        def wait_write(b):
            pltpu.make_async_copy(
                vout[b], out_hbm.at[pl.ds(0, 32)], ssems[b]
            ).wait()

        def transpose(b):
            def body(r, carry):
                for q in range(4):
                    t_loc = r * 4 + q
                    for h in range(2):
                        vals = plsc.load_gather(
                            vin[b],
                            [iota + h * LANES,
                             jnp.full((LANES,), t_loc, jnp.int32)])
                        vout[b][r, pl.ds(q * 32 + h * LANES, LANES)] = vals
                return carry
            lax.fori_loop(0, 32, body, 0)

        valid = (TCOLS - 1 - wid) // NW + 1

        for b in range(DG):
            @pl.when(b < valid)
            def _(b=b):
                start_read(b, b)

        def body(k, carry):
            for bb in range(NBUF):
                @pl.when(lax.rem(k, NBUF) == bb)
                def _(bb=bb):
                    nxt = k + DG
                    bn = (bb + DG) % NBUF

                    @pl.when(nxt < valid)
                    def _():
                        @pl.when(nxt >= NBUF)
                        def _():
                            wait_write(bn)
                        start_read(nxt, bn)

                    wait_read(bb)
                    transpose(bb)
                    start_write(k, bb)
            return carry

        lax.fori_loop(0, valid, body, 0)

        for j in range(NBUF):
            @pl.when(valid > j)
            def _(j=j):
                for bb in range(NBUF):
                    @pl.when(lax.rem(valid - 1 - j, NBUF) == bb)
                    def _(bb=bb):
                        wait_write(bb)

        if TAIL:
            @pl.when(wid == 0)
            def _():
                pltpu.async_copy(
                    tokT_hbm.at[:, pl.ds(TCOLS * 128, TAIL)], tin, gsems[0]
                ).wait()

                def tail_body(r, carry):
                    for q in range(4):
                        t_loc = r * 4 + q
                        for h in range(2):
                            vals = plsc.load_gather(
                                tin,
                                [iota + h * LANES,
                                 jnp.full((LANES,), t_loc, jnp.int32)])
                            tout[r, pl.ds(q * 32 + h * LANES, LANES)] = vals
                    return carry
                lax.fori_loop(0, TAIL * 32 // 128, tail_body, 0)
                pltpu.async_copy(
                    tout, out_hbm.at[pl.ds(TCOLS * 32, TAIL * 32 // 128)],
                    gsems[0],
                ).wait()

    return detile


@functools.lru_cache(maxsize=None)
def _make_emb(batch, maxlen, embed, vocab):
    RPW = batch // NW
    EB = embed // 8
    assert RPW == 128 and embed == 32 and maxlen % NBUF == 0

    mesh = plsc.VectorSubcoreMesh(core_axis_name="c", subcore_axis_name="s")

    @functools.partial(
        pl.kernel,
        mesh=mesh,
        compiler_params=pltpu.CompilerParams(use_tc_tiling_on_sc=False, needs_layout_passes=False),
        out_type=jax.ShapeDtypeStruct((maxlen, EB, batch // 128, 8, 128),
                                      jnp.float32),
        scratch_types=(
            [pltpu.VMEM((RPW, maxlen), jnp.int32),
             pltpu.VMEM((maxlen, RPW), jnp.int32),
             pltpu.VMEM((maxlen, embed), jnp.float32)]
            + [pltpu.VMEM((RPW, embed), jnp.float32) for _ in range(NBUF)]
            + [pltpu.VMEM((EB, 1, 8, 128), jnp.float32) for _ in range(NBUF)]
            + [pltpu.SemaphoreType.DMA for _ in range(2 * NBUF + 1)]
        ),
    )
    def emb(x_hbm, tok_hbm, pos_hbm, out_hbm, idx_v, idxT_v, pat_v, *rest):
        gbuf = rest[:NBUF]
        obuf = rest[NBUF:2 * NBUF]
        gsems = rest[2 * NBUF:3 * NBUF]
        ssems = rest[3 * NBUF:4 * NBUF]
        lsem = rest[4 * NBUF]

        wid = lax.axis_index("s") * NC + lax.axis_index("c")
        base = wid * RPW
        iota = _iota16()

        pltpu.async_copy(x_hbm.at[pl.ds(base, RPW)], idx_v, lsem).wait()
        pltpu.async_copy(pos_hbm, pat_v, lsem).wait()

        def tbody(t, carry):
            for j in range(RPW // LANES):
                vals = plsc.load_gather(
                    idx_v, [j * LANES + iota,
                            jnp.full((LANES,), t, jnp.int32)])
                idxT_v[t, pl.ds(j * LANES, LANES)] = vals
            return carry
        lax.fori_loop(0, maxlen, tbody, 0)

        def start_gather(t, b):
            pltpu.make_async_copy(
                tok_hbm.at[idxT_v.at[t]], gbuf[b], gsems[b]
            ).start()

        def wait_gather(b):
            pltpu.make_async_copy(
                tok_hbm.at[idxT_v.at[0]], gbuf[b], gsems[b]
            ).wait()

        def start_scatter(t, b):
            pltpu.make_async_copy(
                obuf[b], out_hbm.at[t, :, pl.ds(wid, 1)], ssems[b]
            ).start()

        def wait_scatter(b):
            pltpu.make_async_copy(
                obuf[b], out_hbm.at[0, :, pl.ds(wid, 1)], ssems[b]
            ).wait()

        def transpose_add(t, b):
            def ebody(te, carry):
                for r in range(8):
                    e = te * 8 + r
                    pv = plsc.load_gather(
                        pat_v, [jnp.full((LANES,), t, jnp.int32),
                                jnp.full((LANES,), e, jnp.int32)])
                    for g in range(RPW // LANES):
                        vals = plsc.load_gather(
                            gbuf[b], [g * LANES + iota,
                                      jnp.full((LANES,), e, jnp.int32)])
                        obuf[b][te, 0, r, pl.ds(g * LANES, LANES)] = vals + pv
                return carry
            lax.fori_loop(0, EB, ebody, 0)

        for b in range(DG):
            start_gather(b, b)

        def outer(i, carry):
            t0 = i * NBUF
            for b in range(NBUF):
                t = t0 + b
                nxt = t + DG
                bn = (b + DG) % NBUF

                @pl.when(nxt < maxlen)
                def _(nxt=nxt, bn=bn):
                    @pl.when(nxt >= NBUF)
                    def _():
                        wait_scatter(bn)
                    start_gather(nxt, bn)

                wait_gather(b)
                transpose_add(t, b)
                start_scatter(t, b)
            return carry

        lax.fori_loop(0, maxlen // NBUF, outer, 0)

        for b in range(NBUF):
            wait_scatter(b)

    return emb


def kernel(x, token_table, pos_table):
    batch, maxlen = x.shape
    vocab, embed = token_table.shape
    tok_dense = _make_detile(vocab, embed)(token_table.T)
    tok_lin = tok_dense.reshape(vocab, embed)
    out5 = _make_emb(batch, maxlen, embed, vocab)(
        x.astype(jnp.int32), tok_lin, pos_table
    )
    return out5.transpose(2, 4, 0, 1, 3).reshape(batch, maxlen, embed)


# speedup vs baseline: 2.6857x; 2.6857x over previous
import functools

import jax
import jax.numpy as jnp
from jax import lax
from jax.experimental import pallas as pl
from jax.experimental.pallas import tpu as pltpu
from jax.experimental.pallas import tpu_sc as plsc

LANES = 16
NC = 2
NS = 16
NW = NC * NS
NBUF = 4
DG = 2


def _iota16():
    return lax.iota(jnp.int32, LANES)


@functools.lru_cache(maxsize=None)
def _make_detile(vocab, embed):
    TCOLS = vocab // 128
    TAIL = vocab % 128
    assert embed == 32 and TAIL % 4 == 0
    mesh = plsc.VectorSubcoreMesh(core_axis_name="c", subcore_axis_name="s")

    @functools.partial(
        pl.kernel,
        mesh=mesh,
        compiler_params=pltpu.CompilerParams(needs_layout_passes=False),
        out_type=jax.ShapeDtypeStruct((vocab * embed // 128, 128), jnp.float32),
        scratch_types=(
            [pltpu.VMEM((32, 128), jnp.float32) for _ in range(2 * NBUF)]
            + ([pltpu.VMEM((32, TAIL), jnp.float32),
                pltpu.VMEM((TAIL * 32 // 128, 128), jnp.float32)] if TAIL else [])
            + [pltpu.SemaphoreType.DMA for _ in range(2 * NBUF)]
        ),
    )
    def detile(tokT_hbm, out_hbm, *rest):
        vin = rest[:NBUF]
        vout = rest[NBUF:2 * NBUF]
        ntail = 2 if TAIL else 0
        if TAIL:
            tin, tout = rest[2 * NBUF:2 * NBUF + 2]
        gsems = rest[2 * NBUF + ntail:3 * NBUF + ntail]
        ssems = rest[3 * NBUF + ntail:4 * NBUF + ntail]

        wid = lax.axis_index("s") * NC + lax.axis_index("c")
        iota = _iota16()

        def col_of(k):
            return k * NW + wid

        def start_read(k, b):
            pltpu.make_async_copy(
                tokT_hbm.at[:, pl.ds(col_of(k) * 128, 128)], vin[b], gsems[b]
            ).start()

        def wait_read(b):
            pltpu.make_async_copy(
                tokT_hbm.at[:, pl.ds(0, 128)], vin[b], gsems[b]
            ).wait()

        def start_write(k, b):
            pltpu.make_async_copy(
                vout[b], out_hbm.at[pl.ds(col_of(k) * 32, 32)], ssems[b]
            ).start()

        def wait_write(b):
            pltpu.make_async_copy(
                vout[b], out_hbm.at[pl.ds(0, 32)], ssems[b]
            ).wait()

        def transpose(b):
            def body(r, carry):
                for q in range(4):
                    t_loc = r * 4 + q
                    for h in range(2):
                        vals = plsc.load_gather(
                            vin[b],
                            [iota + h * LANES,
                             jnp.full((LANES,), t_loc, jnp.int32)])
                        vout[b][r, pl.ds(q * 32 + h * LANES, LANES)] = vals
                return carry
            lax.fori_loop(0, 32, body, 0)

        valid = (TCOLS - 1 - wid) // NW + 1

        for b in range(DG):
            @pl.when(b < valid)
            def _(b=b):
                start_read(b, b)

        def body(k, carry):
            for bb in range(NBUF):
                @pl.when(lax.rem(k, NBUF) == bb)
                def _(bb=bb):
                    nxt = k + DG
                    bn = (bb + DG) % NBUF

                    @pl.when(nxt < valid)
                    def _():
                        @pl.when(nxt >= NBUF)
                        def _():
                            wait_write(bn)
                        start_read(nxt, bn)

                    wait_read(bb)
                    transpose(bb)
                    start_write(k, bb)
            return carry

        lax.fori_loop(0, valid, body, 0)

        for j in range(NBUF):
            @pl.when(valid > j)
            def _(j=j):
                for bb in range(NBUF):
                    @pl.when(lax.rem(valid - 1 - j, NBUF) == bb)
                    def _(bb=bb):
                        wait_write(bb)

        if TAIL:
            @pl.when(wid == 0)
            def _():
                pltpu.async_copy(
                    tokT_hbm.at[:, pl.ds(TCOLS * 128, TAIL)], tin, gsems[0]
                ).wait()

                def tail_body(r, carry):
                    for q in range(4):
                        t_loc = r * 4 + q
                        for h in range(2):
                            vals = plsc.load_gather(
                                tin,
                                [iota + h * LANES,
                                 jnp.full((LANES,), t_loc, jnp.int32)])
                            tout[r, pl.ds(q * 32 + h * LANES, LANES)] = vals
                    return carry
                lax.fori_loop(0, TAIL * 32 // 128, tail_body, 0)
                pltpu.async_copy(
                    tout, out_hbm.at[pl.ds(TCOLS * 32, TAIL * 32 // 128)],
                    gsems[0],
                ).wait()

    return detile


@functools.lru_cache(maxsize=None)
def _make_emb(batch, maxlen, embed, vocab):
    RPW = batch // NW
    EB = embed // 8
    assert RPW == 128 and embed == 32 and maxlen % NBUF == 0

    mesh = plsc.VectorSubcoreMesh(core_axis_name="c", subcore_axis_name="s")

    @functools.partial(
        pl.kernel,
        mesh=mesh,
        compiler_params=pltpu.CompilerParams(use_tc_tiling_on_sc=False, needs_layout_passes=False),
        out_type=jax.ShapeDtypeStruct((maxlen, EB, batch // 128, 8, 128),
                                      jnp.float32),
        scratch_types=(
            [pltpu.VMEM((RPW, maxlen), jnp.int32),
             pltpu.VMEM((maxlen, RPW), jnp.int32),
             pltpu.VMEM((maxlen, embed), jnp.float32)]
            + [pltpu.VMEM((RPW, embed), jnp.float32) for _ in range(NBUF)]
            + [pltpu.VMEM((EB, 1, 8, 128), jnp.float32) for _ in range(NBUF)]
            + [pltpu.SemaphoreType.DMA for _ in range(2 * NBUF + 1)]
        ),
    )
    def emb(x_hbm, tok_hbm, pos_hbm, out_hbm, idx_v, idxT_v, pat_v, *rest):
        gbuf = rest[:NBUF]
        obuf = rest[NBUF:2 * NBUF]
        gsems = rest[2 * NBUF:3 * NBUF]
        ssems = rest[3 * NBUF:4 * NBUF]
        lsem = rest[4 * NBUF]

        wid = lax.axis_index("s") * NC + lax.axis_index("c")
        base = wid * RPW
        iota = _iota16()

        pltpu.async_copy(x_hbm.at[pl.ds(base, RPW)], idx_v, lsem).wait()
        pltpu.async_copy(pos_hbm, pat_v, lsem).wait()

        def tbody(t, carry):
            for j in range(RPW // LANES):
                vals = plsc.load_gather(
                    idx_v, [j * LANES + iota,
                            jnp.full((LANES,), t, jnp.int32)])
                idxT_v[t, pl.ds(j * LANES, LANES)] = vals
            return carry
        lax.fori_loop(0, maxlen, tbody, 0)

        def start_gather(t, b):
            pltpu.make_async_copy(
                tok_hbm.at[idxT_v.at[t]], gbuf[b], gsems[b]
            ).start()

        def wait_gather(b):
            pltpu.make_async_copy(
                tok_hbm.at[idxT_v.at[0]], gbuf[b], gsems[b]
            ).wait()

        def start_scatter(t, b):
            pltpu.make_async_copy(
                obuf[b], out_hbm.at[t, :, pl.ds(wid, 1)], ssems[b]
            ).start()

        def wait_scatter(b):
            pltpu.make_async_copy(
                obuf[b], out_hbm.at[0, :, pl.ds(wid, 1)], ssems[b]
            ).wait()

        def transpose_add(t, b):
            def ebody(te, carry):
                for r in range(8):
                    e = te * 8 + r
                    pv = plsc.load_gather(
                        pat_v, [jnp.full((LANES,), t, jnp.int32),
                                jnp.full((LANES,), e, jnp.int32)])
                    for g in range(RPW // LANES):
                        vals = plsc.load_gather(
                            gbuf[b], [g * LANES + iota,
                                      jnp.full((LANES,), e, jnp.int32)])
                        obuf[b][te, 0, r, pl.ds(g * LANES, LANES)] = vals + pv
                return carry
            lax.fori_loop(0, EB, ebody, 0)

        for b in range(DG):
            start_gather(b, b)

        def outer(i, carry):
            t0 = i * NBUF
            for b in range(NBUF):
                t = t0 + b
                nxt = t + DG
                bn = (b + DG) % NBUF

                @pl.when(nxt < maxlen)
                def _(nxt=nxt, bn=bn):
                    @pl.when(nxt >= NBUF)
                    def _():
                        wait_scatter(bn)
                    start_gather(nxt, bn)

                wait_gather(b)
                transpose_add(t, b)
                start_scatter(t, b)
            return carry

        lax.fori_loop(0, maxlen // NBUF, outer, 0)

        for b in range(NBUF):
            wait_scatter(b)

    return emb


def kernel(x, token_table, pos_table):
    batch, maxlen = x.shape
    vocab, embed = token_table.shape
    tok_dense = _make_detile(vocab, embed)(token_table.T)
    tok_lin = tok_dense.reshape(vocab, embed)
    out5 = _make_emb(batch, maxlen, embed, vocab)(
        x.astype(jnp.int32), tok_lin, pos_table
    )
    return out5.transpose(2, 4, 0, 1, 3).reshape(batch, maxlen, embed)


# speedup vs baseline: 5.1388x; 1.9134x over previous
import functools

import jax
import jax.numpy as jnp
from jax import lax
from jax.experimental import pallas as pl
from jax.experimental.pallas import tpu as pltpu
from jax.experimental.pallas import tpu_sc as plsc

LANES = 16
NC = 2
NS = 16
NW = NC * NS
NBUF = 4
DG = 2


def _iota16():
    return lax.iota(jnp.int32, LANES)


@functools.lru_cache(maxsize=None)
def _make_detile(vocab, embed):
    TCOLS = vocab // 128
    TAIL = vocab % 128
    assert embed == 32 and TAIL % 4 == 0
    mesh = plsc.VectorSubcoreMesh(core_axis_name="c", subcore_axis_name="s")

    @functools.partial(
        pl.kernel,
        mesh=mesh,
        compiler_params=pltpu.CompilerParams(needs_layout_passes=False),
        out_type=jax.ShapeDtypeStruct((vocab * embed // 128, 128), jnp.float32),
        scratch_types=(
            [pltpu.VMEM((32, 128), jnp.float32) for _ in range(2 * NBUF)]
            + ([pltpu.VMEM((32, TAIL), jnp.float32),
                pltpu.VMEM((TAIL * 32 // 128, 128), jnp.float32)] if TAIL else [])
            + [pltpu.SemaphoreType.DMA for _ in range(2 * NBUF)]
        ),
    )
    def detile(tokT_hbm, out_hbm, *rest):
        vin = rest[:NBUF]
        vout = rest[NBUF:2 * NBUF]
        ntail = 2 if TAIL else 0
        if TAIL:
            tin, tout = rest[2 * NBUF:2 * NBUF + 2]
        gsems = rest[2 * NBUF + ntail:3 * NBUF + ntail]
        ssems = rest[3 * NBUF + ntail:4 * NBUF + ntail]

        wid = lax.axis_index("s") * NC + lax.axis_index("c")
        iota = _iota16()

        def col_of(k):
            return k * NW + wid

        def start_read(k, b):
            pltpu.make_async_copy(
                tokT_hbm.at[:, pl.ds(col_of(k) * 128, 128)], vin[b], gsems[b]
            ).start()

        def wait_read(b):
            pltpu.make_async_copy(
                tokT_hbm.at[:, pl.ds(0, 128)], vin[b], gsems[b]
            ).wait()

        def start_write(k, b):
            pltpu.make_async_copy(
                vout[b], out_hbm.at[pl.ds(col_of(k) * 32, 32)], ssems[b]
            ).start()

        def wait_write(b):
            pltpu.make_async_copy(
                vout[b], out_hbm.at[pl.ds(0, 32)], ssems[b]
            ).wait()

        e_lo = iota
        e_hi = iota + LANES

        def transpose(b):
            @plsc.parallel_loop(0, 32, unroll=4)
            def body(r):
                for q in range(4):
                    t_vec = jnp.full((LANES,), r * 4 + q, jnp.int32)
                    for h, e_vec in ((0, e_lo), (1, e_hi)):
                        vals = plsc.load_gather(vin[b], [e_vec, t_vec])
                        vout[b][r, pl.ds(q * 32 + h * LANES, LANES)] = vals

        valid = (TCOLS - 1 - wid) // NW + 1

        for b in range(DG):
            @pl.when(b < valid)
            def _(b=b):
                start_read(b, b)

        def body(k, carry):
            for bb in range(NBUF):
                @pl.when(lax.rem(k, NBUF) == bb)
                def _(bb=bb):
                    nxt = k + DG
                    bn = (bb + DG) % NBUF

                    @pl.when(nxt < valid)
                    def _():
                        @pl.when(nxt >= NBUF)
                        def _():
                            wait_write(bn)
                        start_read(nxt, bn)

                    wait_read(bb)
                    transpose(bb)
                    start_write(k, bb)
            return carry

        lax.fori_loop(0, valid, body, 0)

        for j in range(NBUF):
            @pl.when(valid > j)
            def _(j=j):
                for bb in range(NBUF):
                    @pl.when(lax.rem(valid - 1 - j, NBUF) == bb)
                    def _(bb=bb):
                        wait_write(bb)

        if TAIL:
            @pl.when(wid == 0)
            def _():
                pltpu.async_copy(
                    tokT_hbm.at[:, pl.ds(TCOLS * 128, TAIL)], tin, gsems[0]
                ).wait()

                @plsc.parallel_loop(0, TAIL * 32 // 128, unroll=4)
                def tail_body(r):
                    for q in range(4):
                        t_vec = jnp.full((LANES,), r * 4 + q, jnp.int32)
                        for h, e_vec in ((0, e_lo), (1, e_hi)):
                            vals = plsc.load_gather(tin, [e_vec, t_vec])
                            tout[r, pl.ds(q * 32 + h * LANES, LANES)] = vals
                pltpu.async_copy(
                    tout, out_hbm.at[pl.ds(TCOLS * 32, TAIL * 32 // 128)],
                    gsems[0],
                ).wait()

    return detile


@functools.lru_cache(maxsize=None)
def _make_emb(batch, maxlen, embed, vocab):
    RPW = batch // NW
    EB = embed // 8
    assert RPW == 128 and embed == 32 and maxlen % NBUF == 0

    mesh = plsc.VectorSubcoreMesh(core_axis_name="c", subcore_axis_name="s")

    @functools.partial(
        pl.kernel,
        mesh=mesh,
        compiler_params=pltpu.CompilerParams(use_tc_tiling_on_sc=False, needs_layout_passes=False),
        out_type=jax.ShapeDtypeStruct((maxlen, EB, batch // 128, 8, 128),
                                      jnp.float32),
        scratch_types=(
            [pltpu.VMEM((RPW, maxlen), jnp.int32),
             pltpu.VMEM((maxlen, RPW), jnp.int32),
             pltpu.VMEM((maxlen, embed), jnp.float32)]
            + [pltpu.VMEM((RPW, embed), jnp.float32) for _ in range(NBUF)]
            + [pltpu.VMEM((EB, 1, 8, 128), jnp.float32) for _ in range(NBUF)]
            + [pltpu.SemaphoreType.DMA for _ in range(2 * NBUF + 1)]
        ),
    )
    def emb(x_hbm, tok_hbm, pos_hbm, out_hbm, idx_v, idxT_v, pat_v, *rest):
        gbuf = rest[:NBUF]
        obuf = rest[NBUF:2 * NBUF]
        gsems = rest[2 * NBUF:3 * NBUF]
        ssems = rest[3 * NBUF:4 * NBUF]
        lsem = rest[4 * NBUF]

        wid = lax.axis_index("s") * NC + lax.axis_index("c")
        base = wid * RPW
        iota = _iota16()

        pltpu.async_copy(x_hbm.at[pl.ds(base, RPW)], idx_v, lsem).wait()
        pltpu.async_copy(pos_hbm, pat_v, lsem).wait()

        b_vecs = [j * LANES + iota for j in range(RPW // LANES)]

        @plsc.parallel_loop(0, maxlen, unroll=4)
        def tbody(t):
            t_vec = jnp.full((LANES,), t, jnp.int32)
            for j in range(RPW // LANES):
                vals = plsc.load_gather(idx_v, [b_vecs[j], t_vec])
                idxT_v[t, pl.ds(j * LANES, LANES)] = vals

        def start_gather(t, b):
            pltpu.make_async_copy(
                tok_hbm.at[idxT_v.at[t]], gbuf[b], gsems[b]
            ).start()

        def wait_gather(b):
            pltpu.make_async_copy(
                tok_hbm.at[idxT_v.at[0]], gbuf[b], gsems[b]
            ).wait()

        def start_scatter(t, b):
            pltpu.make_async_copy(
                obuf[b], out_hbm.at[t, :, pl.ds(wid, 1)], ssems[b]
            ).start()

        def wait_scatter(b):
            pltpu.make_async_copy(
                obuf[b], out_hbm.at[0, :, pl.ds(wid, 1)], ssems[b]
            ).wait()

        def transpose_add(t, b):
            t_vec = jnp.full((LANES,), t, jnp.int32)

            @plsc.parallel_loop(0, embed, unroll=4)
            def ebody(e):
                e_vec = jnp.full((LANES,), e, jnp.int32)
                pv = plsc.load_gather(pat_v, [t_vec, e_vec])
                for g in range(RPW // LANES):
                    vals = plsc.load_gather(gbuf[b], [b_vecs[g], e_vec])
                    obuf[b][e >> 3, 0, e & 7, pl.ds(g * LANES, LANES)] = (
                        vals + pv)

        for b in range(DG):
            start_gather(b, b)

        def outer(i, carry):
            t0 = i * NBUF
            for b in range(NBUF):
                t = t0 + b
                nxt = t + DG
                bn = (b + DG) % NBUF

                @pl.when(nxt < maxlen)
                def _(nxt=nxt, bn=bn):
                    @pl.when(nxt >= NBUF)
                    def _():
                        wait_scatter(bn)
                    start_gather(nxt, bn)

                wait_gather(b)
                transpose_add(t, b)
                start_scatter(t, b)
            return carry

        lax.fori_loop(0, maxlen // NBUF, outer, 0)

        for b in range(NBUF):
            wait_scatter(b)

    return emb


def kernel(x, token_table, pos_table):
    batch, maxlen = x.shape
    vocab, embed = token_table.shape
    tok_dense = _make_detile(vocab, embed)(token_table.T)
    tok_lin = tok_dense.reshape(vocab, embed)
    out5 = _make_emb(batch, maxlen, embed, vocab)(
        x.astype(jnp.int32), tok_lin, pos_table
    )
    return out5.transpose(2, 4, 0, 1, 3).reshape(batch, maxlen, embed)
